# baseline (speedup 1.0000x reference)
import math

import jax
import jax.numpy as jnp
from jax import lax
from jax.experimental import pallas as pl
from jax.experimental.pallas import tpu as pltpu
from jax.experimental.pallas import tpu_sc as plsc

V = 32000
SMOOTH_A = 0.1 / (V - 1)
CONF_C = 0.9
CMA = CONF_C - SMOOTH_A
K_CONST = (V - 1) * SMOOTH_A * math.log(SMOOTH_A) + CONF_C * math.log(CONF_C)

N_ROWS = 4096
R_BLK = 512
V_BLK = 6400
NV = V // V_BLK

R_SC = 512
SC_RB = R_SC // R_BLK
NR_TC = (N_ROWS - R_SC) // R_BLK

SC_NC = 2
SC_NS = 16
SC_NW = SC_NC * SC_NS
LANES = 16
RPW = R_SC // SC_NW
GROUPS = RPW // 8
W_CH = 6400
NCH = V // W_CH
NQ = GROUPS * NCH


def _sum_body(tgt_ref, pred_ref, out_ref, acc_ref, cnt_ref):
    i = pl.program_id(0)
    j = pl.program_id(1)

    @pl.when((i == 0) & (j == 0))
    def _init():
        acc_ref[0] = 0.0
        cnt_ref[0] = 0.0

    tgt = tgt_ref[0, 0, :]
    maskf = (tgt > 0).astype(jnp.float32)

    @pl.when(j == 0)
    def _count():
        cnt_ref[0] += jnp.sum(maskf)

    pred = pred_ref[...]
    tloc = tgt - j * V_BLK
    col0 = jax.lax.broadcasted_iota(jnp.int32, (R_BLK, V_BLK), 1)
    w = jnp.where(col0 == tloc[:, None], CONF_C, SMOOTH_A)
    row_part = jnp.sum(pred * w, axis=1)
    acc_ref[0] += jnp.sum(row_part * maskf)

    @pl.when((i == NR_TC - 1) & (j == NV - 1))
    def _fin():
        out_ref[0] = acc_ref[0]
        out_ref[1] = cnt_ref[0]


def _tc_partial(pred, tgt3):
    return pl.pallas_call(
        _sum_body,
        grid=(NR_TC, NV),
        compiler_params=pltpu.CompilerParams(
            vmem_limit_bytes=100 * 1024 * 1024),
        in_specs=[
            pl.BlockSpec((1, 1, R_BLK), lambda i, j: (i + SC_RB, 0, 0)),
            pl.BlockSpec((R_BLK, V_BLK), lambda i, j: (i + SC_RB, j)),
        ],
        out_specs=pl.BlockSpec(memory_space=pltpu.SMEM),
        out_shape=jax.ShapeDtypeStruct((2,), jnp.float32),
        scratch_shapes=[
            pltpu.SMEM((1,), jnp.float32),
            pltpu.SMEM((1,), jnp.float32),
        ],
    )(tgt3, pred)


def _sc_body(pred_hbm, tgtb_hbm, out_m, out_c, tgtb_v, buf0, buf1, acc_v,
             sem0, sem1):
    wid = lax.axis_index("s") * SC_NC + lax.axis_index("c")
    r_base = wid * RPW
    pltpu.sync_copy(tgtb_hbm.at[pl.ds(r_base, RPW), :], tgtb_v)
    iot = lax.iota(jnp.int32, LANES)

    def src(q):
        g = q // NCH
        c0 = (q % NCH) * W_CH
        return pred_hbm.at[pl.ds(r_base + g * 8, 8), pl.ds(c0, W_CH)]

    pltpu.async_copy(src(0), buf0, sem0)
    pltpu.async_copy(src(1), buf1, sem1)

    NACC = 8

    def chunk(q, buf, money, cntv):
        g = q // NCH
        c0 = (q % NCH) * W_CH
        first = (q % NCH) == 0
        for s in range(8):
            l = g * 8 + s
            tsp = tgtb_v[l, pl.ds(0, LANES)]
            maskv = jnp.where(tsp > 0, 1.0, 0.0)

            def vsum(i, carry):
                accs = list(carry[:NACC])
                g0, g1 = carry[NACC], carry[NACC + 1]
                base = i * (NACC * LANES)
                for k in range(NACC):
                    v = buf[s, pl.ds(base + k * LANES, LANES)]
                    colv = (c0 + base + k * LANES) + iot
                    accs[k] = accs[k] + v
                    hitv = jnp.where(colv == tsp, v, 0.0)
                    if k % 2 == 0:
                        g0 = g0 + hitv
                    else:
                        g1 = g1 + hitv
                return tuple(accs) + (g0, g1)

            z = jnp.zeros((LANES,), jnp.float32)
            out = lax.fori_loop(0, W_CH // (NACC * LANES), vsum,
                                (z,) * (NACC + 2), unroll=2)
            rs = out[0]
            for k in range(1, NACC):
                rs = rs + out[k]
            gv = out[NACC] + out[NACC + 1]
            money = money + (rs * SMOOTH_A + gv * CMA) * maskv
            firstf = jnp.where(first, 1.0, 0.0)
            lane0 = jnp.where(iot == 0, 1.0, 0.0)
            cntv = cntv + lane0 * maskv * firstf
        return money, cntv

    def pair(q2, carry):
        money, cntv = carry
        q = q2 * 2
        pltpu.make_async_copy(src(q), buf0, sem0).wait()
        money, cntv = chunk(q, buf0, money, cntv)

        @pl.when(q + 2 < NQ)
        def _():
            pltpu.async_copy(src(q + 2), buf0, sem0)

        pltpu.make_async_copy(src(q + 1), buf1, sem1).wait()
        money, cntv = chunk(q + 1, buf1, money, cntv)

        @pl.when(q + 3 < NQ)
        def _():
            pltpu.async_copy(src(q + 3), buf1, sem1)

        return money, cntv

    z16 = jnp.zeros((LANES,), jnp.float32)
    money, cntv = lax.fori_loop(0, NQ // 2, pair, (z16, z16))

    acc_v[...] = money
    pltpu.sync_copy(acc_v, out_m.at[wid])
    acc_v[...] = cntv
    pltpu.sync_copy(acc_v, out_c.at[wid])


def _sc_partial(pred, tgt_bcast):
    run = pl.kernel(
        _sc_body,
        out_type=[
            jax.ShapeDtypeStruct((SC_NW, LANES), jnp.float32),
            jax.ShapeDtypeStruct((SC_NW, LANES), jnp.float32),
        ],
        mesh=plsc.VectorSubcoreMesh(core_axis_name="c", subcore_axis_name="s"),
        scratch_types=[
            pltpu.VMEM((RPW, 128), jnp.int32),
            pltpu.VMEM((8, W_CH), jnp.float32),
            pltpu.VMEM((8, W_CH), jnp.float32),
            pltpu.VMEM((LANES,), jnp.float32),
            pltpu.SemaphoreType.DMA,
            pltpu.SemaphoreType.DMA,
        ],
    )
    return run(pred, tgt_bcast)


def kernel(prediction, target):
    pred = prediction.reshape(N_ROWS, V)
    tgt_flat = target.reshape(N_ROWS).astype(jnp.int32)
    tgt3 = tgt_flat.reshape(N_ROWS // R_BLK, 1, R_BLK)
    tgt_bcast = jnp.broadcast_to(tgt_flat[:R_SC, None], (R_SC, 128))
    sc_m, sc_c = _sc_partial(pred, tgt_bcast)
    sums = _tc_partial(pred, tgt3)
    acc = sums[0] + jnp.sum(sc_m)
    cnt = sums[1] + jnp.sum(sc_c)
    return K_CONST - acc / cnt


# speedup vs baseline: 1.1571x; 1.1571x over previous
import math

import jax
import jax.numpy as jnp
from jax.experimental import pallas as pl
from jax.experimental.pallas import tpu as pltpu

V = 32000
SMOOTH_A = 0.1 / (V - 1)
CONF_C = 0.9
K_CONST = (V - 1) * SMOOTH_A * math.log(SMOOTH_A) + CONF_C * math.log(CONF_C)

R_BLK = 512
V_BLK = 6400
N_ROWS = 4096
HALF = N_ROWS // 2
NR = HALF // R_BLK
NV = V // V_BLK


def _loss_body(tgt_ref, pa_ref, pb_ref, out_ref, acc_ref, cnt_ref):
    i = pl.program_id(0)
    j = pl.program_id(1)

    @pl.when((i == 0) & (j == 0))
    def _init():
        acc_ref[0] = 0.0
        cnt_ref[0] = 0.0

    tgt = tgt_ref[0, 0, :]

    @pl.when(j == 0)
    def _count():
        cnt_ref[0] += jnp.sum((tgt > 0).astype(jnp.float32))

    col0 = jax.lax.broadcasted_iota(jnp.int32, (R_BLK, V_BLK), 1)
    s = 0.0
    for k, ref in ((0, pa_ref), (1, pb_ref)):
        tg = tgt[k * R_BLK:(k + 1) * R_BLK]
        maskf = (tg > 0).astype(jnp.float32)
        tloc = tg - j * V_BLK
        w = jnp.where(col0 == tloc[:, None], CONF_C, SMOOTH_A)
        row_part = jnp.sum(ref[...] * w, axis=1)
        s = s + jnp.sum(row_part * maskf)
    acc_ref[0] += s

    @pl.when((i == NR - 1) & (j == NV - 1))
    def _fin():
        out_ref[0] = K_CONST - acc_ref[0] / cnt_ref[0]


def kernel(prediction, target):
    pred = prediction.reshape(N_ROWS, V)
    tgt = target.reshape(N_ROWS).astype(jnp.int32)
    tgt2 = jnp.concatenate(
        [tgt[:HALF].reshape(NR, 1, R_BLK), tgt[HALF:].reshape(NR, 1, R_BLK)],
        axis=2)
    out = pl.pallas_call(
        _loss_body,
        grid=(NR, NV),
        compiler_params=pltpu.CompilerParams(
            vmem_limit_bytes=100 * 1024 * 1024),
        in_specs=[
            pl.BlockSpec((1, 1, 2 * R_BLK), lambda i, j: (i, 0, 0)),
            pl.BlockSpec((R_BLK, V_BLK), lambda i, j: (i, j)),
            pl.BlockSpec((R_BLK, V_BLK), lambda i, j: (i + NR, j)),
        ],
        out_specs=pl.BlockSpec(memory_space=pltpu.SMEM),
        out_shape=jax.ShapeDtypeStruct((1,), jnp.float32),
        scratch_shapes=[
            pltpu.SMEM((1,), jnp.float32),
            pltpu.SMEM((1,), jnp.float32),
        ],
    )(tgt2, pred, pred)
    return out[0]
